# Initial kernel scaffold; baseline (speedup 1.0000x reference)
import jax
import jax.numpy as jnp
from jax.experimental import pallas as pl


def kernel(Z, R, idx_i, idx_j, Wi0, bi0, Wi, bi, Wo1, bo1, Wo2, bo2, scales, shifts):
    raise NotImplementedError("write your pallas kernel here")


# speedup vs baseline: 22.6568x; 22.6568x over previous
import functools

import jax
import jax.numpy as jnp
import numpy as np
from jax import lax
from jax.experimental import pallas as pl
from jax.experimental.pallas import tpu as pltpu
from jax.experimental.pallas import tpu_sc as plsc

SR_CUT = 6.0
BETA = 0.2

_NC = 2
_NS = 16
_NW = _NC * _NS

_SC_PARAMS = pltpu.CompilerParams(
    use_tc_tiling_on_sc=False, needs_layout_passes=False)


def _sc_mesh():
    return plsc.VectorSubcoreMesh(
        core_axis_name="c", subcore_axis_name="s",
        num_cores=_NC, num_subcores=_NS)


def _make_pair_r2(n_atoms, n_pairs, n_pad, n_sp_pad):
    ppw = n_pairs // _NW
    apw = n_pad // _NW

    @functools.partial(
        pl.kernel,
        out_type=(
            jax.ShapeDtypeStruct((n_pairs,), jnp.float32),
            jax.ShapeDtypeStruct((n_pad,), jnp.float32),
            jax.ShapeDtypeStruct((n_pad,), jnp.float32),
            jax.ShapeDtypeStruct((n_pad,), jnp.float32),
            jax.ShapeDtypeStruct((n_pad,), jnp.float32),
        ),
        mesh=_sc_mesh(),
        compiler_params=_SC_PARAMS,
        scratch_types=[
            pltpu.VMEM((n_atoms,), jnp.float32),
            pltpu.VMEM((n_atoms,), jnp.float32),
            pltpu.VMEM((n_atoms,), jnp.float32),
            pltpu.VMEM((ppw,), jnp.int32),
            pltpu.VMEM((ppw,), jnp.int32),
            pltpu.VMEM((ppw,), jnp.float32),
            pltpu.VMEM((2 * n_sp_pad,), jnp.float32),
            pltpu.VMEM((2 * n_sp_pad,), jnp.float32),
            pltpu.VMEM((apw,), jnp.int32),
            pltpu.VMEM((4 * apw,), jnp.float32),
        ],
    )
    def pair_r2(rx_hbm, ry_hbm, rz_hbm, ii_hbm, jj_hbm, z_hbm, sc_hbm, sh_hbm,
                r2_hbm, sc0_hbm, sc1_hbm, sh0_hbm, sh1_hbm,
                rx_v, ry_v, rz_v, ii_v, jj_v, r2_v, sc_v, sh_v, z_v, g_v):
        wid = lax.axis_index("s") * _NC + lax.axis_index("c")
        base = wid * ppw
        pltpu.sync_copy(rx_hbm, rx_v)
        pltpu.sync_copy(ry_hbm, ry_v)
        pltpu.sync_copy(rz_hbm, rz_v)
        pltpu.sync_copy(ii_hbm.at[pl.ds(base, ppw)], ii_v)
        pltpu.sync_copy(jj_hbm.at[pl.ds(base, ppw)], jj_v)

        def body(k, carry):
            off = k * 16
            iv = ii_v[pl.ds(off, 16)]
            jv = jj_v[pl.ds(off, 16)]
            dx = plsc.load_gather(rx_v, [jv]) - plsc.load_gather(rx_v, [iv])
            dy = plsc.load_gather(ry_v, [jv]) - plsc.load_gather(ry_v, [iv])
            dz = plsc.load_gather(rz_v, [jv]) - plsc.load_gather(rz_v, [iv])
            r2_v[pl.ds(off, 16)] = dx * dx + dy * dy + dz * dz
            return carry

        lax.fori_loop(0, ppw // 16, body, 0)
        pltpu.sync_copy(r2_v, r2_hbm.at[pl.ds(base, ppw)])

        abase = wid * apw
        pltpu.sync_copy(sc_hbm, sc_v)
        pltpu.sync_copy(sh_hbm, sh_v)
        pltpu.sync_copy(z_hbm.at[pl.ds(abase, apw)], z_v)

        def gbody(k, carry):
            off = k * 16
            zv = z_v[pl.ds(off, 16)]
            g_v[pl.ds(off, 16)] = plsc.load_gather(sc_v, [zv])
            g_v[pl.ds(apw + off, 16)] = plsc.load_gather(sc_v, [zv + n_sp_pad])
            g_v[pl.ds(2 * apw + off, 16)] = plsc.load_gather(sh_v, [zv])
            g_v[pl.ds(3 * apw + off, 16)] = (
                plsc.load_gather(sh_v, [zv + n_sp_pad]))
            return carry

        lax.fori_loop(0, apw // 16, gbody, 0)
        pltpu.sync_copy(g_v.at[pl.ds(0, apw)], sc0_hbm.at[pl.ds(abase, apw)])
        pltpu.sync_copy(g_v.at[pl.ds(apw, apw)],
                        sc1_hbm.at[pl.ds(abase, apw)])
        pltpu.sync_copy(g_v.at[pl.ds(2 * apw, apw)],
                        sh0_hbm.at[pl.ds(abase, apw)])
        pltpu.sync_copy(g_v.at[pl.ds(3 * apw, apw)],
                        sh1_hbm.at[pl.ds(abase, apw)])

    return pair_r2


def _ssp(x):
    return jnp.maximum(x, 0.0) + jnp.log1p(jnp.exp(-jnp.abs(x))) - np.log(2.0)


def _mlp_body(num_blocks, k_basis, r2_ref, cen_ref, wit_ref, bit_ref,
              wo1t_ref, bo1t_ref, bd_ref, bo2_ref, rij_ref, *obt_refs):
    r2 = r2_ref[0]
    rij = jnp.sqrt(r2 + 1e-12)
    rij_ref[0] = rij
    fc = jnp.where(rij < SR_CUT,
                   0.5 * (jnp.cos(np.pi / SR_CUT * rij) + 1.0), 0.0)
    e = jnp.exp(-rij)
    cen = cen_ref[...]
    x = fc * jnp.exp(-BETA * (e - cen) ** 2)
    hs = []
    for b in range(num_blocks):
        x = _ssp(jnp.dot(wit_ref[b], x, preferred_element_type=jnp.float32)
                 + bit_ref[b])
        h = _ssp(jnp.dot(wo1t_ref[b], x, preferred_element_type=jnp.float32)
                 + bo1t_ref[b])
        hs.append(h)
    h_cat = jnp.concatenate(hs, axis=0)
    obt = (jnp.dot(bd_ref[...], h_cat, preferred_element_type=jnp.float32)
           + bo2_ref[...])
    for c, ref in enumerate(obt_refs):
        ref[...] = obt[c]


def _make_scatter(n_atoms, n_pairs, nch, n_pad):
    ppw = n_pairs // _NW
    chunk = 2000
    nchunks = ppw // chunk
    acc_n = nch * n_pad

    @functools.partial(
        pl.kernel,
        out_type=jax.ShapeDtypeStruct((_NW * acc_n,), jnp.float32),
        mesh=_sc_mesh(),
        compiler_params=_SC_PARAMS,
        scratch_types=[
            pltpu.VMEM((acc_n,), jnp.float32),
            pltpu.VMEM((nch * chunk,), jnp.float32),
            pltpu.VMEM((chunk,), jnp.int32),
        ],
    )
    def scatter(ob0, ob1, ob2, ob3, ob4, ob5, ob6, ob7, ob8, ob9,
                ii_hbm, part_hbm, acc_v, ob_v, ii_v):
        obs = (ob0, ob1, ob2, ob3, ob4, ob5, ob6, ob7, ob8, ob9)
        wid = lax.axis_index("s") * _NC + lax.axis_index("c")
        base = wid * ppw

        def zbody(k, carry):
            acc_v[pl.ds(k * 16, 16)] = jnp.zeros((16,), jnp.float32)
            return carry

        lax.fori_loop(0, acc_n // 16, zbody, 0)

        stripe = chunk // 16
        lane0 = lax.iota(jnp.int32, 16) * stripe
        for s in range(nchunks):
            pbase = base + s * chunk
            pltpu.sync_copy(ii_hbm.at[pl.ds(pbase, chunk)], ii_v)
            for c in range(nch):
                pltpu.sync_copy(obs[c].at[pl.ds(pbase, chunk)],
                                ob_v.at[pl.ds(c * chunk, chunk)])

            def sbody(k, carry):
                pos = lane0 + k
                iv = plsc.load_gather(ii_v, [pos])
                for c in range(nch):
                    val = plsc.load_gather(ob_v, [pos + c * chunk])
                    plsc.addupdate_scatter(acc_v, [iv + c * n_pad], val)
                return carry

            lax.fori_loop(0, stripe, sbody, 0)

        pltpu.sync_copy(acc_v, part_hbm.at[pl.ds(wid * acc_n, acc_n)])

    return scatter


def _epi_body(num_blocks, n_atoms, n_pad, part_ref, sc0_ref, sc1_ref,
              sh0_ref, sh1_ref, out0_ref, out1_ref, nh_ref, acc_ref):
    w = pl.program_id(0)
    nw = pl.num_programs(0)

    @pl.when(w == 0)
    def _init():
        acc_ref[...] = part_ref[...]

    @pl.when(w != 0)
    def _acc():
        acc_ref[...] = acc_ref[...] + part_ref[...]

    @pl.when(w == nw - 1)
    def _final():
        nch = 2 * num_blocks
        rows = [acc_ref[pl.ds(c * n_pad, n_pad)] for c in range(nch)]
        out0 = rows[0]
        out1 = rows[1]
        for b in range(1, num_blocks):
            out0 = out0 + rows[2 * b]
            out1 = out1 + rows[2 * b + 1]
        nh = jnp.zeros((), jnp.float32)
        for b in range(1, num_blocks):
            for o in range(2):
                x2 = rows[2 * b + o] * rows[2 * b + o]
                p2 = rows[2 * (b - 1) + o] * rows[2 * (b - 1) + o]
                nh = nh + jnp.sum(x2 / (x2 + p2 + 1e-07)) / (2.0 * n_atoms)
        nh_ref[...] = jnp.reshape(nh, (1, 1))
        out0_ref[...] = out0 * sc0_ref[...] + sh0_ref[...]
        out1_ref[...] = out1 * sc1_ref[...] + sh1_ref[...]


def kernel(Z, R, idx_i, idx_j, Wi0, bi0, Wi, bi, Wo1, bo1, Wo2, bo2,
           scales, shifts):
    n_atoms = R.shape[0]
    n_pairs = idx_i.shape[0]
    k_basis = Wi0.shape[0]
    num_blocks = Wo1.shape[0]
    n_out = Wo2.shape[2]
    n_sp = scales.shape[1]
    nch = num_blocks * n_out
    n_pad = 10240
    n_sp_pad = 96

    ii = idx_i.astype(jnp.int32)
    jj = idx_j.astype(jnp.int32)
    rx = R[:, 0]
    ry = R[:, 1]
    rz = R[:, 2]
    zpad = jnp.pad(Z.astype(jnp.int32), (0, n_pad - n_atoms))
    scpad = jnp.pad(scales, ((0, 0), (0, n_sp_pad - n_sp))).reshape(-1)
    shpad = jnp.pad(shifts, ((0, 0), (0, n_sp_pad - n_sp))).reshape(-1)

    r2, sc0, sc1, sh0, sh1 = _make_pair_r2(n_atoms, n_pairs, n_pad, n_sp_pad)(
        rx, ry, rz, ii, jj, zpad, scpad, shpad)

    w_in = jnp.concatenate([Wi0[None], Wi], axis=0)
    wit = jnp.transpose(w_in, (0, 2, 1))
    bit = jnp.concatenate([bi0[None], bi], axis=0)[:, :, None]
    wo1t = jnp.transpose(Wo1, (0, 2, 1))
    bo1t = bo1[:, :, None]
    bdt = jnp.zeros((nch, num_blocks * k_basis), jnp.float32)
    for b in range(num_blocks):
        bdt = bdt.at[n_out * b:n_out * (b + 1),
                     k_basis * b:k_basis * (b + 1)].set(Wo2[b].T)
    bo2v = bo2.reshape(nch)[:, None]

    pb = 5120
    grid = n_pairs // pb
    centers = jnp.asarray(
        np.linspace(np.exp(-SR_CUT), 1.0, k_basis).astype(np.float32)
    )[:, None]
    outs = pl.pallas_call(
        functools.partial(_mlp_body, num_blocks, k_basis),
        grid=(grid,),
        in_specs=[
            pl.BlockSpec((1, 1, pb), lambda t: (t, 0, 0)),
            pl.BlockSpec((k_basis, 1), lambda t: (0, 0)),
            pl.BlockSpec(wit.shape, lambda t: (0, 0, 0)),
            pl.BlockSpec(bit.shape, lambda t: (0, 0, 0)),
            pl.BlockSpec(wo1t.shape, lambda t: (0, 0, 0)),
            pl.BlockSpec(bo1t.shape, lambda t: (0, 0, 0)),
            pl.BlockSpec(bdt.shape, lambda t: (0, 0)),
            pl.BlockSpec(bo2v.shape, lambda t: (0, 0)),
        ],
        out_specs=[pl.BlockSpec((1, 1, pb), lambda t: (t, 0, 0))] + [
            pl.BlockSpec((pb,), lambda t: (t,)) for _ in range(nch)],
        out_shape=[jax.ShapeDtypeStruct((grid, 1, pb), jnp.float32)] + [
            jax.ShapeDtypeStruct((n_pairs,), jnp.float32)
            for _ in range(nch)],
    )(r2.reshape(grid, 1, pb), centers, wit, bit, wo1t, bo1t, bdt, bo2v)
    rij2d = outs[0]
    obt_list = outs[1:]

    part = _make_scatter(n_atoms, n_pairs, nch, n_pad)(*obt_list, ii)

    acc_n = nch * n_pad
    out0, out1, nh = pl.pallas_call(
        functools.partial(_epi_body, num_blocks, n_atoms, n_pad),
        grid=(_NW,),
        in_specs=[
            pl.BlockSpec((acc_n,), lambda w: (w,)),
            pl.BlockSpec((n_pad,), lambda w: (0,)),
            pl.BlockSpec((n_pad,), lambda w: (0,)),
            pl.BlockSpec((n_pad,), lambda w: (0,)),
            pl.BlockSpec((n_pad,), lambda w: (0,)),
        ],
        out_specs=[
            pl.BlockSpec((n_pad,), lambda w: (0,)),
            pl.BlockSpec((n_pad,), lambda w: (0,)),
            pl.BlockSpec((1, 1), lambda w: (0, 0)),
        ],
        out_shape=[
            jax.ShapeDtypeStruct((n_pad,), jnp.float32),
            jax.ShapeDtypeStruct((n_pad,), jnp.float32),
            jax.ShapeDtypeStruct((1, 1), jnp.float32),
        ],
        scratch_shapes=[pltpu.VMEM((acc_n,), jnp.float32)],
    )(part, sc0, sc1, sh0, sh1)

    outputs = jnp.stack([out0[:n_atoms], out1[:n_atoms]], axis=1)
    rij = rij2d.reshape(n_pairs)
    nhloss = nh.reshape(())
    return (outputs, rij, nhloss)


# speedup vs baseline: 25.1667x; 1.1108x over previous
import functools

import jax
import jax.numpy as jnp
import numpy as np
from jax import lax
from jax.experimental import pallas as pl
from jax.experimental.pallas import tpu as pltpu
from jax.experimental.pallas import tpu_sc as plsc

SR_CUT = 6.0
BETA = 0.2

_NC = 2
_NS = 16
_NW = _NC * _NS

_SC_PARAMS = pltpu.CompilerParams(
    use_tc_tiling_on_sc=False, needs_layout_passes=False)


def _sc_mesh():
    return plsc.VectorSubcoreMesh(
        core_axis_name="c", subcore_axis_name="s",
        num_cores=_NC, num_subcores=_NS)


def _make_pair_r2(n_atoms, n_pairs, n_pad, n_sp_pad):
    ppw = n_pairs // _NW
    apw = n_pad // _NW

    @functools.partial(
        pl.kernel,
        out_type=(
            jax.ShapeDtypeStruct((n_pairs,), jnp.float32),
            jax.ShapeDtypeStruct((n_pad,), jnp.float32),
            jax.ShapeDtypeStruct((n_pad,), jnp.float32),
            jax.ShapeDtypeStruct((n_pad,), jnp.float32),
            jax.ShapeDtypeStruct((n_pad,), jnp.float32),
        ),
        mesh=_sc_mesh(),
        compiler_params=_SC_PARAMS,
        scratch_types=[
            pltpu.VMEM((n_atoms,), jnp.float32),
            pltpu.VMEM((n_atoms,), jnp.float32),
            pltpu.VMEM((n_atoms,), jnp.float32),
            pltpu.VMEM((ppw,), jnp.int32),
            pltpu.VMEM((ppw,), jnp.int32),
            pltpu.VMEM((ppw,), jnp.float32),
            pltpu.VMEM((2 * n_sp_pad,), jnp.float32),
            pltpu.VMEM((2 * n_sp_pad,), jnp.float32),
            pltpu.VMEM((apw,), jnp.int32),
            pltpu.VMEM((4 * apw,), jnp.float32),
            pltpu.SemaphoreType.DMA,
        ],
    )
    def pair_r2(rx_hbm, ry_hbm, rz_hbm, ii_hbm, jj_hbm, z_hbm, sc_hbm, sh_hbm,
                r2_hbm, sc0_hbm, sc1_hbm, sh0_hbm, sh1_hbm,
                rx_v, ry_v, rz_v, ii_v, jj_v, r2_v, sc_v, sh_v, z_v, g_v,
                sem):
        wid = lax.axis_index("s") * _NC + lax.axis_index("c")
        base = wid * ppw
        abase = wid * apw
        pltpu.sync_copy(rx_hbm, rx_v)
        pltpu.sync_copy(ry_hbm, ry_v)
        pltpu.sync_copy(rz_hbm, rz_v)
        pltpu.sync_copy(ii_hbm.at[pl.ds(base, ppw)], ii_v)
        pltpu.sync_copy(jj_hbm.at[pl.ds(base, ppw)], jj_v)
        pltpu.sync_copy(sc_hbm, sc_v)
        pltpu.sync_copy(sh_hbm, sh_v)
        pltpu.sync_copy(z_hbm.at[pl.ds(abase, apw)], z_v)

        def body(k, carry):
            off = k * 16
            iv = ii_v[pl.ds(off, 16)]
            jv = jj_v[pl.ds(off, 16)]
            dx = plsc.load_gather(rx_v, [jv]) - plsc.load_gather(rx_v, [iv])
            dy = plsc.load_gather(ry_v, [jv]) - plsc.load_gather(ry_v, [iv])
            dz = plsc.load_gather(rz_v, [jv]) - plsc.load_gather(rz_v, [iv])
            r2_v[pl.ds(off, 16)] = dx * dx + dy * dy + dz * dz
            return carry

        lax.fori_loop(0, ppw // 16, body, 0)
        pltpu.sync_copy(r2_v, r2_hbm.at[pl.ds(base, ppw)])

        def gbody(k, carry):
            off = k * 16
            zv = z_v[pl.ds(off, 16)]
            g_v[pl.ds(off, 16)] = plsc.load_gather(sc_v, [zv])
            g_v[pl.ds(apw + off, 16)] = plsc.load_gather(sc_v, [zv + n_sp_pad])
            g_v[pl.ds(2 * apw + off, 16)] = plsc.load_gather(sh_v, [zv])
            g_v[pl.ds(3 * apw + off, 16)] = (
                plsc.load_gather(sh_v, [zv + n_sp_pad]))
            return carry

        lax.fori_loop(0, apw // 16, gbody, 0)
        pltpu.sync_copy(g_v.at[pl.ds(0, apw)], sc0_hbm.at[pl.ds(abase, apw)])
        pltpu.sync_copy(g_v.at[pl.ds(apw, apw)],
                        sc1_hbm.at[pl.ds(abase, apw)])
        pltpu.sync_copy(g_v.at[pl.ds(2 * apw, apw)],
                        sh0_hbm.at[pl.ds(abase, apw)])
        pltpu.sync_copy(g_v.at[pl.ds(3 * apw, apw)],
                        sh1_hbm.at[pl.ds(abase, apw)])

    return pair_r2


def _ssp(x):
    return jnp.maximum(x, 0.0) + jnp.log1p(jnp.exp(-jnp.abs(x))) - np.log(2.0)


def _mlp_body(num_blocks, k_basis, r2_ref, cen_ref, wit_ref, bit_ref,
              wo1t_ref, bo1t_ref, bd_ref, bo2_ref, rij_ref, *obt_refs):
    r2 = r2_ref[0]
    rij = jnp.sqrt(r2 + 1e-12)
    rij_ref[0] = rij
    fc = jnp.where(rij < SR_CUT,
                   0.5 * (jnp.cos(np.pi / SR_CUT * rij) + 1.0), 0.0)
    e = jnp.exp(-rij)
    cen = cen_ref[...]
    x = fc * jnp.exp(-BETA * (e - cen) ** 2)
    hs = []
    for b in range(num_blocks):
        x = _ssp(jnp.dot(wit_ref[b], x, preferred_element_type=jnp.float32)
                 + bit_ref[b])
        h = _ssp(jnp.dot(wo1t_ref[b], x, preferred_element_type=jnp.float32)
                 + bo1t_ref[b])
        hs.append(h)
    h_cat = jnp.concatenate(hs, axis=0)
    obt = (jnp.dot(bd_ref[...], h_cat, preferred_element_type=jnp.float32)
           + bo2_ref[...])
    for c, ref in enumerate(obt_refs):
        ref[...] = obt[c]


def _make_scatter(n_atoms, n_pairs, nch, n_pad):
    ppw = n_pairs // _NW
    chunk = 2000
    nchunks = ppw // chunk
    acc_n = nch * n_pad

    @functools.partial(
        pl.kernel,
        out_type=jax.ShapeDtypeStruct((_NW * acc_n,), jnp.float32),
        mesh=_sc_mesh(),
        compiler_params=_SC_PARAMS,
        scratch_types=[
            pltpu.VMEM((acc_n,), jnp.float32),
            pltpu.VMEM((nch * chunk,), jnp.float32),
            pltpu.VMEM((chunk,), jnp.int32),
            pltpu.SemaphoreType.DMA,
        ],
    )
    def scatter(ob0, ob1, ob2, ob3, ob4, ob5, ob6, ob7, ob8, ob9,
                ii_hbm, part_hbm, acc_v, ob_v, ii_v, sem):
        obs = (ob0, ob1, ob2, ob3, ob4, ob5, ob6, ob7, ob8, ob9)
        wid = lax.axis_index("s") * _NC + lax.axis_index("c")
        base = wid * ppw

        def zbody(k, carry):
            acc_v[pl.ds(k * 16, 16)] = jnp.zeros((16,), jnp.float32)
            return carry

        lax.fori_loop(0, acc_n // 16, zbody, 0)

        stripe = chunk // 16
        lane0 = lax.iota(jnp.int32, 16) * stripe
        for s in range(nchunks):
            pbase = base + s * chunk
            cps = [pltpu.async_copy(ii_hbm.at[pl.ds(pbase, chunk)],
                                    ii_v, sem)]
            for c in range(nch):
                cps.append(pltpu.async_copy(
                    obs[c].at[pl.ds(pbase, chunk)],
                    ob_v.at[pl.ds(c * chunk, chunk)], sem))
            for cp in cps:
                cp.wait()

            def sbody(k, carry):
                pos = lane0 + k
                iv = plsc.load_gather(ii_v, [pos])
                for c in range(nch):
                    val = plsc.load_gather(ob_v, [pos + c * chunk])
                    plsc.addupdate_scatter(acc_v, [iv + c * n_pad], val)
                return carry

            lax.fori_loop(0, stripe, sbody, 0)

        pltpu.sync_copy(acc_v, part_hbm.at[pl.ds(wid * acc_n, acc_n)])

    return scatter


def _epi_body(num_blocks, n_atoms, n_pad, part_ref, sc0_ref, sc1_ref,
              sh0_ref, sh1_ref, out0_ref, out1_ref, nh_ref, acc_ref):
    w = pl.program_id(0)
    nw = pl.num_programs(0)

    @pl.when(w == 0)
    def _init():
        acc_ref[...] = part_ref[...]

    @pl.when(w != 0)
    def _acc():
        acc_ref[...] = acc_ref[...] + part_ref[...]

    @pl.when(w == nw - 1)
    def _final():
        nch = 2 * num_blocks
        rows = [acc_ref[pl.ds(c * n_pad, n_pad)] for c in range(nch)]
        out0 = rows[0]
        out1 = rows[1]
        for b in range(1, num_blocks):
            out0 = out0 + rows[2 * b]
            out1 = out1 + rows[2 * b + 1]
        nh = jnp.zeros((), jnp.float32)
        for b in range(1, num_blocks):
            for o in range(2):
                x2 = rows[2 * b + o] * rows[2 * b + o]
                p2 = rows[2 * (b - 1) + o] * rows[2 * (b - 1) + o]
                nh = nh + jnp.sum(x2 / (x2 + p2 + 1e-07)) / (2.0 * n_atoms)
        nh_ref[...] = jnp.reshape(nh, (1, 1))
        out0_ref[...] = out0 * sc0_ref[...] + sh0_ref[...]
        out1_ref[...] = out1 * sc1_ref[...] + sh1_ref[...]


def kernel(Z, R, idx_i, idx_j, Wi0, bi0, Wi, bi, Wo1, bo1, Wo2, bo2,
           scales, shifts):
    n_atoms = R.shape[0]
    n_pairs = idx_i.shape[0]
    k_basis = Wi0.shape[0]
    num_blocks = Wo1.shape[0]
    n_out = Wo2.shape[2]
    n_sp = scales.shape[1]
    nch = num_blocks * n_out
    n_pad = 10240
    n_sp_pad = 96

    ii = idx_i.astype(jnp.int32)
    jj = idx_j.astype(jnp.int32)
    rx = R[:, 0]
    ry = R[:, 1]
    rz = R[:, 2]
    zpad = jnp.pad(Z.astype(jnp.int32), (0, n_pad - n_atoms))
    scpad = jnp.pad(scales, ((0, 0), (0, n_sp_pad - n_sp))).reshape(-1)
    shpad = jnp.pad(shifts, ((0, 0), (0, n_sp_pad - n_sp))).reshape(-1)

    r2, sc0, sc1, sh0, sh1 = _make_pair_r2(n_atoms, n_pairs, n_pad, n_sp_pad)(
        rx, ry, rz, ii, jj, zpad, scpad, shpad)

    w_in = jnp.concatenate([Wi0[None], Wi], axis=0)
    wit = jnp.transpose(w_in, (0, 2, 1))
    bit = jnp.concatenate([bi0[None], bi], axis=0)[:, :, None]
    wo1t = jnp.transpose(Wo1, (0, 2, 1))
    bo1t = bo1[:, :, None]
    bdt = jnp.zeros((nch, num_blocks * k_basis), jnp.float32)
    for b in range(num_blocks):
        bdt = bdt.at[n_out * b:n_out * (b + 1),
                     k_basis * b:k_basis * (b + 1)].set(Wo2[b].T)
    bo2v = bo2.reshape(nch)[:, None]

    pb = 5120
    grid = n_pairs // pb
    centers = jnp.asarray(
        np.linspace(np.exp(-SR_CUT), 1.0, k_basis).astype(np.float32)
    )[:, None]
    outs = pl.pallas_call(
        functools.partial(_mlp_body, num_blocks, k_basis),
        grid=(grid,),
        in_specs=[
            pl.BlockSpec((1, 1, pb), lambda t: (t, 0, 0)),
            pl.BlockSpec((k_basis, 1), lambda t: (0, 0)),
            pl.BlockSpec(wit.shape, lambda t: (0, 0, 0)),
            pl.BlockSpec(bit.shape, lambda t: (0, 0, 0)),
            pl.BlockSpec(wo1t.shape, lambda t: (0, 0, 0)),
            pl.BlockSpec(bo1t.shape, lambda t: (0, 0, 0)),
            pl.BlockSpec(bdt.shape, lambda t: (0, 0)),
            pl.BlockSpec(bo2v.shape, lambda t: (0, 0)),
        ],
        out_specs=[pl.BlockSpec((1, 1, pb), lambda t: (t, 0, 0))] + [
            pl.BlockSpec((pb,), lambda t: (t,)) for _ in range(nch)],
        out_shape=[jax.ShapeDtypeStruct((grid, 1, pb), jnp.float32)] + [
            jax.ShapeDtypeStruct((n_pairs,), jnp.float32)
            for _ in range(nch)],
    )(r2.reshape(grid, 1, pb), centers, wit, bit, wo1t, bo1t, bdt, bo2v)
    rij2d = outs[0]
    obt_list = outs[1:]

    part = _make_scatter(n_atoms, n_pairs, nch, n_pad)(*obt_list, ii)

    acc_n = nch * n_pad
    out0, out1, nh = pl.pallas_call(
        functools.partial(_epi_body, num_blocks, n_atoms, n_pad),
        grid=(_NW,),
        in_specs=[
            pl.BlockSpec((acc_n,), lambda w: (w,)),
            pl.BlockSpec((n_pad,), lambda w: (0,)),
            pl.BlockSpec((n_pad,), lambda w: (0,)),
            pl.BlockSpec((n_pad,), lambda w: (0,)),
            pl.BlockSpec((n_pad,), lambda w: (0,)),
        ],
        out_specs=[
            pl.BlockSpec((n_pad,), lambda w: (0,)),
            pl.BlockSpec((n_pad,), lambda w: (0,)),
            pl.BlockSpec((1, 1), lambda w: (0, 0)),
        ],
        out_shape=[
            jax.ShapeDtypeStruct((n_pad,), jnp.float32),
            jax.ShapeDtypeStruct((n_pad,), jnp.float32),
            jax.ShapeDtypeStruct((1, 1), jnp.float32),
        ],
        scratch_shapes=[pltpu.VMEM((acc_n,), jnp.float32)],
    )(part, sc0, sc1, sh0, sh1)

    outputs = jnp.stack([out0[:n_atoms], out1[:n_atoms]], axis=1)
    rij = rij2d.reshape(n_pairs)
    nhloss = nh.reshape(())
    return (outputs, rij, nhloss)


# speedup vs baseline: 27.6345x; 1.0981x over previous
import functools

import jax
import jax.numpy as jnp
import numpy as np
from jax import lax
from jax.experimental import pallas as pl
from jax.experimental.pallas import tpu as pltpu
from jax.experimental.pallas import tpu_sc as plsc

SR_CUT = 6.0
BETA = 0.2

_NC = 2
_NS = 16
_NW = _NC * _NS

_SC_PARAMS = pltpu.CompilerParams(
    use_tc_tiling_on_sc=False, needs_layout_passes=False)


def _sc_mesh():
    return plsc.VectorSubcoreMesh(
        core_axis_name="c", subcore_axis_name="s",
        num_cores=_NC, num_subcores=_NS)


def _make_pair_r2(n_atoms, n_pairs, n_pad, n_sp_pad):
    ppw = n_pairs // _NW
    apw = n_pad // _NW

    @functools.partial(
        pl.kernel,
        out_type=(
            jax.ShapeDtypeStruct((n_pairs,), jnp.float32),
            jax.ShapeDtypeStruct((n_pad,), jnp.float32),
            jax.ShapeDtypeStruct((n_pad,), jnp.float32),
            jax.ShapeDtypeStruct((n_pad,), jnp.float32),
            jax.ShapeDtypeStruct((n_pad,), jnp.float32),
        ),
        mesh=_sc_mesh(),
        compiler_params=_SC_PARAMS,
        scratch_types=[
            pltpu.VMEM((n_atoms,), jnp.float32),
            pltpu.VMEM((n_atoms,), jnp.float32),
            pltpu.VMEM((n_atoms,), jnp.float32),
            pltpu.VMEM((ppw,), jnp.int32),
            pltpu.VMEM((ppw,), jnp.int32),
            pltpu.VMEM((ppw,), jnp.float32),
            pltpu.VMEM((2 * n_sp_pad,), jnp.float32),
            pltpu.VMEM((2 * n_sp_pad,), jnp.float32),
            pltpu.VMEM((apw,), jnp.int32),
            pltpu.VMEM((4 * apw,), jnp.float32),
            pltpu.SemaphoreType.DMA,
        ],
    )
    def pair_r2(rx_hbm, ry_hbm, rz_hbm, ii_hbm, jj_hbm, z_hbm, sc_hbm, sh_hbm,
                r2_hbm, sc0_hbm, sc1_hbm, sh0_hbm, sh1_hbm,
                rx_v, ry_v, rz_v, ii_v, jj_v, r2_v, sc_v, sh_v, z_v, g_v,
                sem):
        wid = lax.axis_index("s") * _NC + lax.axis_index("c")
        base = wid * ppw
        abase = wid * apw
        pltpu.sync_copy(rx_hbm, rx_v)
        pltpu.sync_copy(ry_hbm, ry_v)
        pltpu.sync_copy(rz_hbm, rz_v)
        pltpu.sync_copy(ii_hbm.at[pl.ds(base, ppw)], ii_v)
        pltpu.sync_copy(jj_hbm.at[pl.ds(base, ppw)], jj_v)
        pltpu.sync_copy(sc_hbm, sc_v)
        pltpu.sync_copy(sh_hbm, sh_v)
        pltpu.sync_copy(z_hbm.at[pl.ds(abase, apw)], z_v)

        def body(k, carry):
            off = k * 16
            iv = ii_v[pl.ds(off, 16)]
            jv = jj_v[pl.ds(off, 16)]
            dx = plsc.load_gather(rx_v, [jv]) - plsc.load_gather(rx_v, [iv])
            dy = plsc.load_gather(ry_v, [jv]) - plsc.load_gather(ry_v, [iv])
            dz = plsc.load_gather(rz_v, [jv]) - plsc.load_gather(rz_v, [iv])
            r2_v[pl.ds(off, 16)] = dx * dx + dy * dy + dz * dz
            return carry

        lax.fori_loop(0, ppw // 16, body, 0)
        pltpu.sync_copy(r2_v, r2_hbm.at[pl.ds(base, ppw)])

        def gbody(k, carry):
            off = k * 16
            zv = z_v[pl.ds(off, 16)]
            g_v[pl.ds(off, 16)] = plsc.load_gather(sc_v, [zv])
            g_v[pl.ds(apw + off, 16)] = plsc.load_gather(sc_v, [zv + n_sp_pad])
            g_v[pl.ds(2 * apw + off, 16)] = plsc.load_gather(sh_v, [zv])
            g_v[pl.ds(3 * apw + off, 16)] = (
                plsc.load_gather(sh_v, [zv + n_sp_pad]))
            return carry

        lax.fori_loop(0, apw // 16, gbody, 0)
        pltpu.sync_copy(g_v.at[pl.ds(0, apw)], sc0_hbm.at[pl.ds(abase, apw)])
        pltpu.sync_copy(g_v.at[pl.ds(apw, apw)],
                        sc1_hbm.at[pl.ds(abase, apw)])
        pltpu.sync_copy(g_v.at[pl.ds(2 * apw, apw)],
                        sh0_hbm.at[pl.ds(abase, apw)])
        pltpu.sync_copy(g_v.at[pl.ds(3 * apw, apw)],
                        sh1_hbm.at[pl.ds(abase, apw)])

    return pair_r2


def _ssp(x):
    return jnp.maximum(x, 0.0) + jnp.log1p(jnp.exp(-jnp.abs(x))) - np.log(2.0)


def _mlp_body(num_blocks, k_basis, r2_ref, cen_ref, wit_ref, bit_ref,
              wo1t_ref, bo1t_ref, bd_ref, bo2_ref, rij_ref, *obt_refs):
    r2 = r2_ref[0]
    rij = jnp.sqrt(r2 + 1e-12)
    rij_ref[0] = rij
    fc = jnp.where(rij < SR_CUT,
                   0.5 * (jnp.cos(np.pi / SR_CUT * rij) + 1.0), 0.0)
    e = jnp.exp(-rij)
    cen = cen_ref[...]
    x = fc * jnp.exp(-BETA * (e - cen) ** 2)
    hs = []
    for b in range(num_blocks):
        x = _ssp(jnp.dot(wit_ref[b], x, preferred_element_type=jnp.float32)
                 + bit_ref[b])
        h = _ssp(jnp.dot(wo1t_ref[b], x, preferred_element_type=jnp.float32)
                 + bo1t_ref[b])
        hs.append(h)
    h_cat = jnp.concatenate(hs, axis=0)
    obt = (jnp.dot(bd_ref[...], h_cat, preferred_element_type=jnp.float32)
           + bo2_ref[...])
    for c, ref in enumerate(obt_refs):
        ref[...] = obt[c]


def _make_scatter(n_atoms, n_pairs, nch, n_pad):
    ppw = n_pairs // _NW
    chunk = 2000
    nchunks = ppw // chunk
    acc_n = nch * n_pad

    @functools.partial(
        pl.kernel,
        out_type=jax.ShapeDtypeStruct((_NW * acc_n,), jnp.float32),
        mesh=_sc_mesh(),
        compiler_params=_SC_PARAMS,
        scratch_types=[
            pltpu.VMEM((acc_n,), jnp.float32),
            pltpu.VMEM((nch * chunk,), jnp.float32),
            pltpu.VMEM((chunk,), jnp.int32),
            pltpu.SemaphoreType.DMA,
        ],
    )
    def scatter(ob0, ob1, ob2, ob3, ob4, ob5, ob6, ob7, ob8, ob9,
                ii_hbm, part_hbm, acc_v, ob_v, ii_v, sem):
        obs = (ob0, ob1, ob2, ob3, ob4, ob5, ob6, ob7, ob8, ob9)
        wid = lax.axis_index("s") * _NC + lax.axis_index("c")
        base = wid * ppw

        def zbody(k, carry):
            acc_v[pl.ds(k * 16, 16)] = jnp.zeros((16,), jnp.float32)
            return carry

        lax.fori_loop(0, acc_n // 16, zbody, 0)

        stripe = chunk // 16
        lane0 = lax.iota(jnp.int32, 16) * stripe
        for s in range(nchunks):
            pbase = base + s * chunk
            cps = [pltpu.async_copy(ii_hbm.at[pl.ds(pbase, chunk)],
                                    ii_v, sem)]
            for c in range(nch):
                cps.append(pltpu.async_copy(
                    obs[c].at[pl.ds(pbase, chunk)],
                    ob_v.at[pl.ds(c * chunk, chunk)], sem))
            for cp in cps:
                cp.wait()

            def sbody(k, carry):
                pos = lane0 + k
                iv = plsc.load_gather(ii_v, [pos])
                for c in range(nch):
                    val = plsc.load_gather(ob_v, [pos + c * chunk])
                    plsc.addupdate_scatter(acc_v, [iv + c * n_pad], val)
                return carry

            lax.fori_loop(0, stripe, sbody, 0)

        pltpu.sync_copy(acc_v, part_hbm.at[pl.ds(wid * acc_n, acc_n)])

    return scatter


def _epi_body(num_blocks, n_atoms, n_pad, part_ref, sc0_ref, sc1_ref,
              sh0_ref, sh1_ref, out0_ref, out1_ref, nh_ref, acc_ref):
    w = pl.program_id(0)
    nw = pl.num_programs(0)

    @pl.when(w == 0)
    def _init():
        acc_ref[...] = part_ref[...]

    @pl.when(w != 0)
    def _acc():
        acc_ref[...] = acc_ref[...] + part_ref[...]

    @pl.when(w == nw - 1)
    def _final():
        nch = 2 * num_blocks
        rows = [acc_ref[pl.ds(c * n_pad, n_pad)] for c in range(nch)]
        out0 = rows[0]
        out1 = rows[1]
        for b in range(1, num_blocks):
            out0 = out0 + rows[2 * b]
            out1 = out1 + rows[2 * b + 1]
        nh = jnp.zeros((), jnp.float32)
        for b in range(1, num_blocks):
            for o in range(2):
                x2 = rows[2 * b + o] * rows[2 * b + o]
                p2 = rows[2 * (b - 1) + o] * rows[2 * (b - 1) + o]
                nh = nh + jnp.sum(x2 / (x2 + p2 + 1e-07)) / (2.0 * n_atoms)
        nh_ref[...] = jnp.reshape(nh, (1, 1))
        out0_ref[...] = out0 * sc0_ref[...] + sh0_ref[...]
        out1_ref[...] = out1 * sc1_ref[...] + sh1_ref[...]


def kernel(Z, R, idx_i, idx_j, Wi0, bi0, Wi, bi, Wo1, bo1, Wo2, bo2,
           scales, shifts):
    n_atoms = R.shape[0]
    n_pairs = idx_i.shape[0]
    k_basis = Wi0.shape[0]
    num_blocks = Wo1.shape[0]
    n_out = Wo2.shape[2]
    n_sp = scales.shape[1]
    nch = num_blocks * n_out
    n_pad = 10240
    n_sp_pad = 96

    ii = idx_i.astype(jnp.int32)
    jj = idx_j.astype(jnp.int32)
    rx = R[:, 0]
    ry = R[:, 1]
    rz = R[:, 2]
    zpad = jnp.pad(Z.astype(jnp.int32), (0, n_pad - n_atoms))
    scpad = jnp.pad(scales, ((0, 0), (0, n_sp_pad - n_sp))).reshape(-1)
    shpad = jnp.pad(shifts, ((0, 0), (0, n_sp_pad - n_sp))).reshape(-1)

    r2, sc0, sc1, sh0, sh1 = _make_pair_r2(n_atoms, n_pairs, n_pad, n_sp_pad)(
        rx, ry, rz, ii, jj, zpad, scpad, shpad)

    w_in = jnp.concatenate([Wi0[None], Wi], axis=0)
    wit = jnp.transpose(w_in, (0, 2, 1))
    bit = jnp.concatenate([bi0[None], bi], axis=0)[:, :, None]
    wo1t = jnp.transpose(Wo1, (0, 2, 1))
    bo1t = bo1[:, :, None]
    bdt = jnp.zeros((nch, num_blocks * k_basis), jnp.float32)
    for b in range(num_blocks):
        bdt = bdt.at[n_out * b:n_out * (b + 1),
                     k_basis * b:k_basis * (b + 1)].set(Wo2[b].T)
    bo2v = bo2.reshape(nch)[:, None]

    pb = 25600
    grid = n_pairs // pb
    centers = jnp.asarray(
        np.linspace(np.exp(-SR_CUT), 1.0, k_basis).astype(np.float32)
    )[:, None]
    outs = pl.pallas_call(
        functools.partial(_mlp_body, num_blocks, k_basis),
        grid=(grid,),
        in_specs=[
            pl.BlockSpec((1, 1, pb), lambda t: (t, 0, 0)),
            pl.BlockSpec((k_basis, 1), lambda t: (0, 0)),
            pl.BlockSpec(wit.shape, lambda t: (0, 0, 0)),
            pl.BlockSpec(bit.shape, lambda t: (0, 0, 0)),
            pl.BlockSpec(wo1t.shape, lambda t: (0, 0, 0)),
            pl.BlockSpec(bo1t.shape, lambda t: (0, 0, 0)),
            pl.BlockSpec(bdt.shape, lambda t: (0, 0)),
            pl.BlockSpec(bo2v.shape, lambda t: (0, 0)),
        ],
        out_specs=[pl.BlockSpec((1, 1, pb), lambda t: (t, 0, 0))] + [
            pl.BlockSpec((pb,), lambda t: (t,)) for _ in range(nch)],
        out_shape=[jax.ShapeDtypeStruct((grid, 1, pb), jnp.float32)] + [
            jax.ShapeDtypeStruct((n_pairs,), jnp.float32)
            for _ in range(nch)],
    )(r2.reshape(grid, 1, pb), centers, wit, bit, wo1t, bo1t, bdt, bo2v)
    rij2d = outs[0]
    obt_list = outs[1:]

    part = _make_scatter(n_atoms, n_pairs, nch, n_pad)(*obt_list, ii)

    acc_n = nch * n_pad
    out0, out1, nh = pl.pallas_call(
        functools.partial(_epi_body, num_blocks, n_atoms, n_pad),
        grid=(_NW,),
        in_specs=[
            pl.BlockSpec((acc_n,), lambda w: (w,)),
            pl.BlockSpec((n_pad,), lambda w: (0,)),
            pl.BlockSpec((n_pad,), lambda w: (0,)),
            pl.BlockSpec((n_pad,), lambda w: (0,)),
            pl.BlockSpec((n_pad,), lambda w: (0,)),
        ],
        out_specs=[
            pl.BlockSpec((n_pad,), lambda w: (0,)),
            pl.BlockSpec((n_pad,), lambda w: (0,)),
            pl.BlockSpec((1, 1), lambda w: (0, 0)),
        ],
        out_shape=[
            jax.ShapeDtypeStruct((n_pad,), jnp.float32),
            jax.ShapeDtypeStruct((n_pad,), jnp.float32),
            jax.ShapeDtypeStruct((1, 1), jnp.float32),
        ],
        scratch_shapes=[pltpu.VMEM((acc_n,), jnp.float32)],
    )(part, sc0, sc1, sh0, sh1)

    outputs = jnp.stack([out0[:n_atoms], out1[:n_atoms]], axis=1)
    rij = rij2d.reshape(n_pairs)
    nhloss = nh.reshape(())
    return (outputs, rij, nhloss)


# speedup vs baseline: 27.9183x; 1.0103x over previous
import functools

import jax
import jax.numpy as jnp
import numpy as np
from jax import lax
from jax.experimental import pallas as pl
from jax.experimental.pallas import tpu as pltpu
from jax.experimental.pallas import tpu_sc as plsc

SR_CUT = 6.0
BETA = 0.2

_NC = 2
_NS = 16
_NW = _NC * _NS

_SC_PARAMS = pltpu.CompilerParams(
    use_tc_tiling_on_sc=False, needs_layout_passes=False)


def _sc_mesh():
    return plsc.VectorSubcoreMesh(
        core_axis_name="c", subcore_axis_name="s",
        num_cores=_NC, num_subcores=_NS)


def _make_pair_r2(n_atoms, n_pairs, n_pad, n_sp_pad):
    ppw = n_pairs // _NW
    apw = n_pad // _NW

    @functools.partial(
        pl.kernel,
        out_type=(
            jax.ShapeDtypeStruct((n_pairs,), jnp.float32),
            jax.ShapeDtypeStruct((n_pad,), jnp.float32),
            jax.ShapeDtypeStruct((n_pad,), jnp.float32),
            jax.ShapeDtypeStruct((n_pad,), jnp.float32),
            jax.ShapeDtypeStruct((n_pad,), jnp.float32),
        ),
        mesh=_sc_mesh(),
        compiler_params=_SC_PARAMS,
        scratch_types=[
            pltpu.VMEM((n_atoms,), jnp.float32),
            pltpu.VMEM((n_atoms,), jnp.float32),
            pltpu.VMEM((n_atoms,), jnp.float32),
            pltpu.VMEM((ppw,), jnp.int32),
            pltpu.VMEM((ppw,), jnp.int32),
            pltpu.VMEM((ppw,), jnp.float32),
            pltpu.VMEM((2 * n_sp_pad,), jnp.float32),
            pltpu.VMEM((2 * n_sp_pad,), jnp.float32),
            pltpu.VMEM((apw,), jnp.int32),
            pltpu.VMEM((4 * apw,), jnp.float32),
            pltpu.SemaphoreType.DMA,
        ],
    )
    def pair_r2(rx_hbm, ry_hbm, rz_hbm, ii_hbm, jj_hbm, z_hbm, sc_hbm, sh_hbm,
                r2_hbm, sc0_hbm, sc1_hbm, sh0_hbm, sh1_hbm,
                rx_v, ry_v, rz_v, ii_v, jj_v, r2_v, sc_v, sh_v, z_v, g_v,
                sem):
        wid = lax.axis_index("s") * _NC + lax.axis_index("c")
        base = wid * ppw
        abase = wid * apw
        pltpu.sync_copy(rx_hbm, rx_v)
        pltpu.sync_copy(ry_hbm, ry_v)
        pltpu.sync_copy(rz_hbm, rz_v)
        pltpu.sync_copy(ii_hbm.at[pl.ds(base, ppw)], ii_v)
        pltpu.sync_copy(jj_hbm.at[pl.ds(base, ppw)], jj_v)
        pltpu.sync_copy(sc_hbm, sc_v)
        pltpu.sync_copy(sh_hbm, sh_v)
        pltpu.sync_copy(z_hbm.at[pl.ds(abase, apw)], z_v)

        def body(k, carry):
            off = k * 16
            iv = ii_v[pl.ds(off, 16)]
            jv = jj_v[pl.ds(off, 16)]
            dx = plsc.load_gather(rx_v, [jv]) - plsc.load_gather(rx_v, [iv])
            dy = plsc.load_gather(ry_v, [jv]) - plsc.load_gather(ry_v, [iv])
            dz = plsc.load_gather(rz_v, [jv]) - plsc.load_gather(rz_v, [iv])
            r2_v[pl.ds(off, 16)] = dx * dx + dy * dy + dz * dz
            return carry

        lax.fori_loop(0, ppw // 16, body, 0)
        pltpu.sync_copy(r2_v, r2_hbm.at[pl.ds(base, ppw)])

        def gbody(k, carry):
            off = k * 16
            zv = z_v[pl.ds(off, 16)]
            g_v[pl.ds(off, 16)] = plsc.load_gather(sc_v, [zv])
            g_v[pl.ds(apw + off, 16)] = plsc.load_gather(sc_v, [zv + n_sp_pad])
            g_v[pl.ds(2 * apw + off, 16)] = plsc.load_gather(sh_v, [zv])
            g_v[pl.ds(3 * apw + off, 16)] = (
                plsc.load_gather(sh_v, [zv + n_sp_pad]))
            return carry

        lax.fori_loop(0, apw // 16, gbody, 0)
        pltpu.sync_copy(g_v.at[pl.ds(0, apw)], sc0_hbm.at[pl.ds(abase, apw)])
        pltpu.sync_copy(g_v.at[pl.ds(apw, apw)],
                        sc1_hbm.at[pl.ds(abase, apw)])
        pltpu.sync_copy(g_v.at[pl.ds(2 * apw, apw)],
                        sh0_hbm.at[pl.ds(abase, apw)])
        pltpu.sync_copy(g_v.at[pl.ds(3 * apw, apw)],
                        sh1_hbm.at[pl.ds(abase, apw)])

    return pair_r2


def _ssp(x):
    return jnp.maximum(x, 0.0) + jnp.log1p(jnp.exp(-jnp.abs(x))) - np.log(2.0)


def _mlp_body(num_blocks, k_basis, r2_ref, cen_ref, wit_ref, bit_ref,
              wo1t_ref, bo1t_ref, bd_ref, bo2_ref, rij_ref, *obt_refs):
    r2 = r2_ref[0]
    rij = jnp.sqrt(r2 + 1e-12)
    rij_ref[0] = rij
    fc = jnp.where(rij < SR_CUT,
                   0.5 * (jnp.cos(np.pi / SR_CUT * rij) + 1.0), 0.0)
    e = jnp.exp(-rij)
    cen = cen_ref[...]
    x = fc * jnp.exp(-BETA * (e - cen) ** 2)
    hs = []
    for b in range(num_blocks):
        x = _ssp(jnp.dot(wit_ref[b], x, preferred_element_type=jnp.float32)
                 + bit_ref[b])
        h = _ssp(jnp.dot(wo1t_ref[b], x, preferred_element_type=jnp.float32)
                 + bo1t_ref[b])
        hs.append(h)
    h_cat = jnp.concatenate(hs, axis=0)
    obt = (jnp.dot(bd_ref[...], h_cat, preferred_element_type=jnp.float32)
           + bo2_ref[...])
    for c, ref in enumerate(obt_refs):
        ref[...] = obt[c]


def _make_scatter(n_atoms, n_pairs, nch, n_pad):
    ppw = n_pairs // _NW
    chunk = 800
    nchunks = ppw // chunk
    acc_n = nch * n_pad

    @functools.partial(
        pl.kernel,
        out_type=jax.ShapeDtypeStruct((_NW * acc_n,), jnp.float32),
        mesh=_sc_mesh(),
        compiler_params=_SC_PARAMS,
        scratch_types=[
            pltpu.VMEM((acc_n,), jnp.float32),
            pltpu.VMEM((nch * chunk,), jnp.float32),
            pltpu.VMEM((nch * chunk,), jnp.float32),
            pltpu.VMEM((chunk,), jnp.int32),
            pltpu.VMEM((chunk,), jnp.int32),
            pltpu.SemaphoreType.DMA,
            pltpu.SemaphoreType.DMA,
        ],
    )
    def scatter(ob0, ob1, ob2, ob3, ob4, ob5, ob6, ob7, ob8, ob9,
                ii_hbm, part_hbm, acc_v, ob_a, ob_b, ii_a, ii_b,
                sem_a, sem_b):
        obs = (ob0, ob1, ob2, ob3, ob4, ob5, ob6, ob7, ob8, ob9)
        slots = ((ob_a, ii_a, sem_a), (ob_b, ii_b, sem_b))
        wid = lax.axis_index("s") * _NC + lax.axis_index("c")
        base = wid * ppw

        def start(s):
            ob_v, ii_v, sem = slots[s % 2]
            pbase = base + s * chunk
            cps = [pltpu.async_copy(ii_hbm.at[pl.ds(pbase, chunk)],
                                    ii_v, sem)]
            for c in range(nch):
                cps.append(pltpu.async_copy(
                    obs[c].at[pl.ds(pbase, chunk)],
                    ob_v.at[pl.ds(c * chunk, chunk)], sem))
            return cps

        pending = start(0)

        def zbody(k, carry):
            acc_v[pl.ds(k * 16, 16)] = jnp.zeros((16,), jnp.float32)
            return carry

        lax.fori_loop(0, acc_n // 16, zbody, 0)

        stripe = chunk // 16
        lane0 = lax.iota(jnp.int32, 16) * stripe
        for s in range(nchunks):
            ob_v, ii_v, _ = slots[s % 2]
            nxt = start(s + 1) if s + 1 < nchunks else []
            for cp in pending:
                cp.wait()
            pending = nxt

            def sbody(k, carry, ob_v=ob_v, ii_v=ii_v):
                pos = lane0 + k
                iv = plsc.load_gather(ii_v, [pos])
                for c in range(nch):
                    val = plsc.load_gather(ob_v, [pos + c * chunk])
                    plsc.addupdate_scatter(acc_v, [iv + c * n_pad], val)
                return carry

            lax.fori_loop(0, stripe, sbody, 0)

        pltpu.sync_copy(acc_v, part_hbm.at[pl.ds(wid * acc_n, acc_n)])

    return scatter


def _epi_body(num_blocks, n_atoms, n_pad, part_ref, sc0_ref, sc1_ref,
              sh0_ref, sh1_ref, out0_ref, out1_ref, nh_ref, acc_ref):
    w = pl.program_id(0)
    nw = pl.num_programs(0)

    @pl.when(w == 0)
    def _init():
        acc_ref[...] = part_ref[...]

    @pl.when(w != 0)
    def _acc():
        acc_ref[...] = acc_ref[...] + part_ref[...]

    @pl.when(w == nw - 1)
    def _final():
        nch = 2 * num_blocks
        rows = [acc_ref[pl.ds(c * n_pad, n_pad)] for c in range(nch)]
        out0 = rows[0]
        out1 = rows[1]
        for b in range(1, num_blocks):
            out0 = out0 + rows[2 * b]
            out1 = out1 + rows[2 * b + 1]
        nh = jnp.zeros((), jnp.float32)
        for b in range(1, num_blocks):
            for o in range(2):
                x2 = rows[2 * b + o] * rows[2 * b + o]
                p2 = rows[2 * (b - 1) + o] * rows[2 * (b - 1) + o]
                nh = nh + jnp.sum(x2 / (x2 + p2 + 1e-07)) / (2.0 * n_atoms)
        nh_ref[...] = jnp.reshape(nh, (1, 1))
        out0_ref[...] = out0 * sc0_ref[...] + sh0_ref[...]
        out1_ref[...] = out1 * sc1_ref[...] + sh1_ref[...]


def kernel(Z, R, idx_i, idx_j, Wi0, bi0, Wi, bi, Wo1, bo1, Wo2, bo2,
           scales, shifts):
    n_atoms = R.shape[0]
    n_pairs = idx_i.shape[0]
    k_basis = Wi0.shape[0]
    num_blocks = Wo1.shape[0]
    n_out = Wo2.shape[2]
    n_sp = scales.shape[1]
    nch = num_blocks * n_out
    n_pad = 10240
    n_sp_pad = 96

    ii = idx_i.astype(jnp.int32)
    jj = idx_j.astype(jnp.int32)
    rx = R[:, 0]
    ry = R[:, 1]
    rz = R[:, 2]
    zpad = jnp.pad(Z.astype(jnp.int32), (0, n_pad - n_atoms))
    scpad = jnp.pad(scales, ((0, 0), (0, n_sp_pad - n_sp))).reshape(-1)
    shpad = jnp.pad(shifts, ((0, 0), (0, n_sp_pad - n_sp))).reshape(-1)

    r2, sc0, sc1, sh0, sh1 = _make_pair_r2(n_atoms, n_pairs, n_pad, n_sp_pad)(
        rx, ry, rz, ii, jj, zpad, scpad, shpad)

    w_in = jnp.concatenate([Wi0[None], Wi], axis=0)
    wit = jnp.transpose(w_in, (0, 2, 1))
    bit = jnp.concatenate([bi0[None], bi], axis=0)[:, :, None]
    wo1t = jnp.transpose(Wo1, (0, 2, 1))
    bo1t = bo1[:, :, None]
    bdt = jnp.zeros((nch, num_blocks * k_basis), jnp.float32)
    for b in range(num_blocks):
        bdt = bdt.at[n_out * b:n_out * (b + 1),
                     k_basis * b:k_basis * (b + 1)].set(Wo2[b].T)
    bo2v = bo2.reshape(nch)[:, None]

    pb = 25600
    grid = n_pairs // pb
    centers = jnp.asarray(
        np.linspace(np.exp(-SR_CUT), 1.0, k_basis).astype(np.float32)
    )[:, None]
    outs = pl.pallas_call(
        functools.partial(_mlp_body, num_blocks, k_basis),
        grid=(grid,),
        in_specs=[
            pl.BlockSpec((1, 1, pb), lambda t: (t, 0, 0)),
            pl.BlockSpec((k_basis, 1), lambda t: (0, 0)),
            pl.BlockSpec(wit.shape, lambda t: (0, 0, 0)),
            pl.BlockSpec(bit.shape, lambda t: (0, 0, 0)),
            pl.BlockSpec(wo1t.shape, lambda t: (0, 0, 0)),
            pl.BlockSpec(bo1t.shape, lambda t: (0, 0, 0)),
            pl.BlockSpec(bdt.shape, lambda t: (0, 0)),
            pl.BlockSpec(bo2v.shape, lambda t: (0, 0)),
        ],
        out_specs=[pl.BlockSpec((1, 1, pb), lambda t: (t, 0, 0))] + [
            pl.BlockSpec((pb,), lambda t: (t,)) for _ in range(nch)],
        out_shape=[jax.ShapeDtypeStruct((grid, 1, pb), jnp.float32)] + [
            jax.ShapeDtypeStruct((n_pairs,), jnp.float32)
            for _ in range(nch)],
    )(r2.reshape(grid, 1, pb), centers, wit, bit, wo1t, bo1t, bdt, bo2v)
    rij2d = outs[0]
    obt_list = outs[1:]

    part = _make_scatter(n_atoms, n_pairs, nch, n_pad)(*obt_list, ii)

    acc_n = nch * n_pad
    out0, out1, nh = pl.pallas_call(
        functools.partial(_epi_body, num_blocks, n_atoms, n_pad),
        grid=(_NW,),
        in_specs=[
            pl.BlockSpec((acc_n,), lambda w: (w,)),
            pl.BlockSpec((n_pad,), lambda w: (0,)),
            pl.BlockSpec((n_pad,), lambda w: (0,)),
            pl.BlockSpec((n_pad,), lambda w: (0,)),
            pl.BlockSpec((n_pad,), lambda w: (0,)),
        ],
        out_specs=[
            pl.BlockSpec((n_pad,), lambda w: (0,)),
            pl.BlockSpec((n_pad,), lambda w: (0,)),
            pl.BlockSpec((1, 1), lambda w: (0, 0)),
        ],
        out_shape=[
            jax.ShapeDtypeStruct((n_pad,), jnp.float32),
            jax.ShapeDtypeStruct((n_pad,), jnp.float32),
            jax.ShapeDtypeStruct((1, 1), jnp.float32),
        ],
        scratch_shapes=[pltpu.VMEM((acc_n,), jnp.float32)],
    )(part, sc0, sc1, sh0, sh1)

    outputs = jnp.stack([out0[:n_atoms], out1[:n_atoms]], axis=1)
    rij = rij2d.reshape(n_pairs)
    nhloss = nh.reshape(())
    return (outputs, rij, nhloss)


# speedup vs baseline: 29.8161x; 1.0680x over previous
import functools

import jax
import jax.numpy as jnp
import numpy as np
from jax import lax
from jax.experimental import pallas as pl
from jax.experimental.pallas import tpu as pltpu
from jax.experimental.pallas import tpu_sc as plsc

SR_CUT = 6.0
BETA = 0.2

_NC = 2
_NS = 16
_NW = _NC * _NS

_SC_PARAMS = pltpu.CompilerParams(
    use_tc_tiling_on_sc=False, needs_layout_passes=False)


def _sc_mesh():
    return plsc.VectorSubcoreMesh(
        core_axis_name="c", subcore_axis_name="s",
        num_cores=_NC, num_subcores=_NS)


def _make_pair_r2(n_atoms, n_pairs, n_pad, n_sp_pad):
    ppw = n_pairs // _NW
    apw = n_pad // _NW

    @functools.partial(
        pl.kernel,
        out_type=(
            jax.ShapeDtypeStruct((n_pairs,), jnp.float32),
            jax.ShapeDtypeStruct((n_pad,), jnp.float32),
            jax.ShapeDtypeStruct((n_pad,), jnp.float32),
            jax.ShapeDtypeStruct((n_pad,), jnp.float32),
            jax.ShapeDtypeStruct((n_pad,), jnp.float32),
        ),
        mesh=_sc_mesh(),
        compiler_params=_SC_PARAMS,
        scratch_types=[
            pltpu.VMEM((n_atoms,), jnp.float32),
            pltpu.VMEM((n_atoms,), jnp.float32),
            pltpu.VMEM((n_atoms,), jnp.float32),
            pltpu.VMEM((ppw,), jnp.int32),
            pltpu.VMEM((ppw,), jnp.int32),
            pltpu.VMEM((ppw,), jnp.float32),
            pltpu.VMEM((2 * n_sp_pad,), jnp.float32),
            pltpu.VMEM((2 * n_sp_pad,), jnp.float32),
            pltpu.VMEM((apw,), jnp.int32),
            pltpu.VMEM((4 * apw,), jnp.float32),
            pltpu.SemaphoreType.DMA,
        ],
    )
    def pair_r2(rx_hbm, ry_hbm, rz_hbm, ii_hbm, jj_hbm, z_hbm, sc_hbm, sh_hbm,
                r2_hbm, sc0_hbm, sc1_hbm, sh0_hbm, sh1_hbm,
                rx_v, ry_v, rz_v, ii_v, jj_v, r2_v, sc_v, sh_v, z_v, g_v,
                sem):
        wid = lax.axis_index("s") * _NC + lax.axis_index("c")
        base = wid * ppw
        abase = wid * apw
        pltpu.sync_copy(rx_hbm, rx_v)
        pltpu.sync_copy(ry_hbm, ry_v)
        pltpu.sync_copy(rz_hbm, rz_v)
        pltpu.sync_copy(ii_hbm.at[pl.ds(base, ppw)], ii_v)
        pltpu.sync_copy(jj_hbm.at[pl.ds(base, ppw)], jj_v)
        pltpu.sync_copy(sc_hbm, sc_v)
        pltpu.sync_copy(sh_hbm, sh_v)
        pltpu.sync_copy(z_hbm.at[pl.ds(abase, apw)], z_v)

        def body(k, carry):
            off = k * 16
            iv = ii_v[pl.ds(off, 16)]
            jv = jj_v[pl.ds(off, 16)]
            dx = plsc.load_gather(rx_v, [jv]) - plsc.load_gather(rx_v, [iv])
            dy = plsc.load_gather(ry_v, [jv]) - plsc.load_gather(ry_v, [iv])
            dz = plsc.load_gather(rz_v, [jv]) - plsc.load_gather(rz_v, [iv])
            r2_v[pl.ds(off, 16)] = dx * dx + dy * dy + dz * dz
            return carry

        lax.fori_loop(0, ppw // 16, body, 0)
        pltpu.sync_copy(r2_v, r2_hbm.at[pl.ds(base, ppw)])

        def gbody(k, carry):
            off = k * 16
            zv = z_v[pl.ds(off, 16)]
            g_v[pl.ds(off, 16)] = plsc.load_gather(sc_v, [zv])
            g_v[pl.ds(apw + off, 16)] = plsc.load_gather(sc_v, [zv + n_sp_pad])
            g_v[pl.ds(2 * apw + off, 16)] = plsc.load_gather(sh_v, [zv])
            g_v[pl.ds(3 * apw + off, 16)] = (
                plsc.load_gather(sh_v, [zv + n_sp_pad]))
            return carry

        lax.fori_loop(0, apw // 16, gbody, 0)
        pltpu.sync_copy(g_v.at[pl.ds(0, apw)], sc0_hbm.at[pl.ds(abase, apw)])
        pltpu.sync_copy(g_v.at[pl.ds(apw, apw)],
                        sc1_hbm.at[pl.ds(abase, apw)])
        pltpu.sync_copy(g_v.at[pl.ds(2 * apw, apw)],
                        sh0_hbm.at[pl.ds(abase, apw)])
        pltpu.sync_copy(g_v.at[pl.ds(3 * apw, apw)],
                        sh1_hbm.at[pl.ds(abase, apw)])

    return pair_r2


def _ssp(x):
    return jnp.maximum(x, 0.0) + jnp.log1p(jnp.exp(-jnp.abs(x))) - np.log(2.0)


def _mlp_body(num_blocks, k_basis, r2_ref, cen_ref, wit_ref, bit_ref,
              wo1t_ref, bo1t_ref, bd_ref, bo2_ref, rij_ref, *obt_refs):
    r2 = r2_ref[0]
    rij = jnp.sqrt(r2 + 1e-12)
    rij_ref[0] = rij
    fc = jnp.where(rij < SR_CUT,
                   0.5 * (jnp.cos(np.pi / SR_CUT * rij) + 1.0), 0.0)
    e = jnp.exp(-rij)
    cen = cen_ref[...]
    x = fc * jnp.exp(-BETA * (e - cen) ** 2)
    hs = []
    for b in range(num_blocks):
        x = _ssp(jnp.dot(wit_ref[b], x, preferred_element_type=jnp.float32)
                 + bit_ref[b])
        h = _ssp(jnp.dot(wo1t_ref[b], x, preferred_element_type=jnp.float32)
                 + bo1t_ref[b])
        hs.append(h)
    h_cat = jnp.concatenate(hs, axis=0)
    obt = (jnp.dot(bd_ref[...], h_cat, preferred_element_type=jnp.float32)
           + bo2_ref[...])
    for c, ref in enumerate(obt_refs):
        ref[...] = obt[c]


def _make_scatter(n_atoms, n_pairs, nch, n_pad):
    ppw = n_pairs // _NW
    chunk = 800
    nchunks = ppw // chunk
    acc_n = nch * n_pad

    @functools.partial(
        pl.kernel,
        out_type=jax.ShapeDtypeStruct((_NW * acc_n,), jnp.float32),
        mesh=_sc_mesh(),
        compiler_params=_SC_PARAMS,
        scratch_types=[
            pltpu.VMEM((acc_n,), jnp.float32),
            pltpu.VMEM((nch * chunk,), jnp.float32),
            pltpu.VMEM((nch * chunk,), jnp.float32),
            pltpu.VMEM((chunk,), jnp.int32),
            pltpu.VMEM((chunk,), jnp.int32),
            pltpu.SemaphoreType.DMA,
            pltpu.SemaphoreType.DMA,
        ],
    )
    def scatter(ob0, ob1, ob2, ob3, ob4, ob5, ob6, ob7, ob8, ob9,
                ii_hbm, part_hbm, acc_v, ob_a, ob_b, ii_a, ii_b,
                sem_a, sem_b):
        obs = (ob0, ob1, ob2, ob3, ob4, ob5, ob6, ob7, ob8, ob9)
        slots = ((ob_a, ii_a, sem_a), (ob_b, ii_b, sem_b))
        wid = lax.axis_index("s") * _NC + lax.axis_index("c")
        base = wid * ppw

        def start(s):
            ob_v, ii_v, sem = slots[s % 2]
            pbase = base + s * chunk
            cps = [pltpu.async_copy(ii_hbm.at[pl.ds(pbase, chunk)],
                                    ii_v, sem)]
            for c in range(nch):
                cps.append(pltpu.async_copy(
                    obs[c].at[pl.ds(pbase, chunk)],
                    ob_v.at[pl.ds(c * chunk, chunk)], sem))
            return cps

        pending = start(0)

        def zbody(k, carry):
            acc_v[pl.ds(k * 16, 16)] = jnp.zeros((16,), jnp.float32)
            return carry

        lax.fori_loop(0, acc_n // 16, zbody, 0)

        stripe = chunk // 16
        lane0 = lax.iota(jnp.int32, 16) * stripe
        for s in range(nchunks):
            ob_v, ii_v, _ = slots[s % 2]
            nxt = start(s + 1) if s + 1 < nchunks else []
            for cp in pending:
                cp.wait()
            pending = nxt

            def sbody(k, carry, ob_v=ob_v, ii_v=ii_v):
                pos = lane0 + k
                iv = plsc.load_gather(ii_v, [pos])
                for c in range(nch):
                    val = plsc.load_gather(ob_v, [pos + c * chunk])
                    plsc.addupdate_scatter(acc_v, [iv + c * n_pad], val)
                return carry

            lax.fori_loop(0, stripe, sbody, 0)

        pltpu.sync_copy(acc_v, part_hbm.at[pl.ds(wid * acc_n, acc_n)])

    return scatter


def _epi_body(num_blocks, n_atoms, n_pad, part_ref, part2_ref, sc0_ref,
              sc1_ref, sh0_ref, sh1_ref, out0_ref, out1_ref, nh_ref,
              acc_ref):
    w = pl.program_id(0)
    nw = pl.num_programs(0)

    @pl.when(w == 0)
    def _init():
        acc_ref[...] = part_ref[...] + part2_ref[...]

    @pl.when(w != 0)
    def _acc():
        acc_ref[...] = acc_ref[...] + part_ref[...] + part2_ref[...]

    @pl.when(w == nw - 1)
    def _final():
        nch = 2 * num_blocks
        rows = [acc_ref[pl.ds(c * n_pad, n_pad)] for c in range(nch)]
        out0 = rows[0]
        out1 = rows[1]
        for b in range(1, num_blocks):
            out0 = out0 + rows[2 * b]
            out1 = out1 + rows[2 * b + 1]
        nh = jnp.zeros((), jnp.float32)
        for b in range(1, num_blocks):
            for o in range(2):
                x2 = rows[2 * b + o] * rows[2 * b + o]
                p2 = rows[2 * (b - 1) + o] * rows[2 * (b - 1) + o]
                nh = nh + jnp.sum(x2 / (x2 + p2 + 1e-07)) / (2.0 * n_atoms)
        nh_ref[...] = jnp.reshape(nh, (1, 1))
        out0_ref[...] = out0 * sc0_ref[...] + sh0_ref[...]
        out1_ref[...] = out1 * sc1_ref[...] + sh1_ref[...]


def kernel(Z, R, idx_i, idx_j, Wi0, bi0, Wi, bi, Wo1, bo1, Wo2, bo2,
           scales, shifts):
    n_atoms = R.shape[0]
    n_pairs = idx_i.shape[0]
    k_basis = Wi0.shape[0]
    num_blocks = Wo1.shape[0]
    n_out = Wo2.shape[2]
    n_sp = scales.shape[1]
    nch = num_blocks * n_out
    n_pad = 10240
    n_sp_pad = 96

    ii = idx_i.astype(jnp.int32)
    jj = idx_j.astype(jnp.int32)
    rx = R[:, 0]
    ry = R[:, 1]
    rz = R[:, 2]
    zpad = jnp.pad(Z.astype(jnp.int32), (0, n_pad - n_atoms))
    scpad = jnp.pad(scales, ((0, 0), (0, n_sp_pad - n_sp))).reshape(-1)
    shpad = jnp.pad(shifts, ((0, 0), (0, n_sp_pad - n_sp))).reshape(-1)

    r2, sc0, sc1, sh0, sh1 = _make_pair_r2(n_atoms, n_pairs, n_pad, n_sp_pad)(
        rx, ry, rz, ii, jj, zpad, scpad, shpad)

    w_in = jnp.concatenate([Wi0[None], Wi], axis=0)
    wit = jnp.transpose(w_in, (0, 2, 1))
    bit = jnp.concatenate([bi0[None], bi], axis=0)[:, :, None]
    wo1t = jnp.transpose(Wo1, (0, 2, 1))
    bo1t = bo1[:, :, None]
    bdt = jnp.zeros((nch, num_blocks * k_basis), jnp.float32)
    for b in range(num_blocks):
        bdt = bdt.at[n_out * b:n_out * (b + 1),
                     k_basis * b:k_basis * (b + 1)].set(Wo2[b].T)
    bo2v = bo2.reshape(nch)[:, None]

    pb = 25600
    centers = jnp.asarray(
        np.linspace(np.exp(-SR_CUT), 1.0, k_basis).astype(np.float32)
    )[:, None]

    def mlp_part(r2_part):
        np_part = r2_part.shape[0]
        grid = np_part // pb
        return pl.pallas_call(
            functools.partial(_mlp_body, num_blocks, k_basis),
            grid=(grid,),
            in_specs=[
                pl.BlockSpec((1, 1, pb), lambda t: (t, 0, 0)),
                pl.BlockSpec((k_basis, 1), lambda t: (0, 0)),
                pl.BlockSpec(wit.shape, lambda t: (0, 0, 0)),
                pl.BlockSpec(bit.shape, lambda t: (0, 0, 0)),
                pl.BlockSpec(wo1t.shape, lambda t: (0, 0, 0)),
                pl.BlockSpec(bo1t.shape, lambda t: (0, 0, 0)),
                pl.BlockSpec(bdt.shape, lambda t: (0, 0)),
                pl.BlockSpec(bo2v.shape, lambda t: (0, 0)),
            ],
            out_specs=[pl.BlockSpec((1, 1, pb), lambda t: (t, 0, 0))] + [
                pl.BlockSpec((pb,), lambda t: (t,)) for _ in range(nch)],
            out_shape=[jax.ShapeDtypeStruct((grid, 1, pb), jnp.float32)] + [
                jax.ShapeDtypeStruct((np_part,), jnp.float32)
                for _ in range(nch)],
        )(r2_part.reshape(grid, 1, pb), centers, wit, bit, wo1t, bo1t,
          bdt, bo2v)

    p1 = 409600
    outs1 = mlp_part(r2[:p1])
    outs2 = mlp_part(r2[p1:])

    part1 = _make_scatter(n_atoms, p1, nch, n_pad)(*outs1[1:], ii[:p1])
    part2 = _make_scatter(n_atoms, n_pairs - p1, nch, n_pad)(
        *outs2[1:], ii[p1:])

    acc_n = nch * n_pad
    out0, out1, nh = pl.pallas_call(
        functools.partial(_epi_body, num_blocks, n_atoms, n_pad),
        grid=(_NW,),
        in_specs=[
            pl.BlockSpec((acc_n,), lambda w: (w,)),
            pl.BlockSpec((acc_n,), lambda w: (w,)),
            pl.BlockSpec((n_pad,), lambda w: (0,)),
            pl.BlockSpec((n_pad,), lambda w: (0,)),
            pl.BlockSpec((n_pad,), lambda w: (0,)),
            pl.BlockSpec((n_pad,), lambda w: (0,)),
        ],
        out_specs=[
            pl.BlockSpec((n_pad,), lambda w: (0,)),
            pl.BlockSpec((n_pad,), lambda w: (0,)),
            pl.BlockSpec((1, 1), lambda w: (0, 0)),
        ],
        out_shape=[
            jax.ShapeDtypeStruct((n_pad,), jnp.float32),
            jax.ShapeDtypeStruct((n_pad,), jnp.float32),
            jax.ShapeDtypeStruct((1, 1), jnp.float32),
        ],
        scratch_shapes=[pltpu.VMEM((acc_n,), jnp.float32)],
    )(part1, part2, sc0, sc1, sh0, sh1)

    outputs = jnp.stack([out0[:n_atoms], out1[:n_atoms]], axis=1)
    rij = jnp.concatenate(
        [outs1[0].reshape(p1), outs2[0].reshape(n_pairs - p1)])
    nhloss = nh.reshape(())
    return (outputs, rij, nhloss)
